# baseline (speedup 1.0000x reference)
import functools

import jax
import jax.numpy as jnp
from jax import lax
from jax.experimental import pallas as pl
from jax.experimental.pallas import tpu as pltpu
from jax.experimental.pallas import tpu_sc as plsc

_NC = 2
_NS = 16
_NW = _NC * _NS

_D = 128
_TP = 56
_NBUF = 4


def _make_gather(S: int):
    assert S % _NW == 0
    s_per_w = S // _NW
    assert s_per_w % _NBUF == 0 and s_per_w // _NBUF >= 2
    n_groups = s_per_w // _NBUF
    mesh = plsc.VectorSubcoreMesh(core_axis_name="c", subcore_axis_name="s")

    @functools.partial(
        pl.kernel,
        mesh=mesh,
        out_type=jax.ShapeDtypeStruct((S, _TP, _D), jnp.float32),
        scratch_types=[
            pltpu.VMEM((s_per_w * _TP,), jnp.int32),
            pltpu.VMEM((_NBUF, _TP, _D), jnp.float32),
            pltpu.SemaphoreType.DMA((_NBUF,)),
            pltpu.SemaphoreType.DMA((_NBUF,)),
        ],
    )
    def k(table_hbm, idx_hbm, out_hbm, idx_v, rows_v, gsem, osem):
        wid = lax.axis_index("s") * _NC + lax.axis_index("c")
        base = wid * s_per_w
        pltpu.sync_copy(
            idx_hbm.at[pl.ds(base * _TP, s_per_w * _TP)], idx_v
        )

        def start_gather(j, b):
            pltpu.async_copy(
                table_hbm.at[idx_v.at[pl.ds(j * _TP, _TP)]],
                rows_v.at[b],
                gsem.at[b],
            )

        def wait_gather(b):
            pltpu.make_async_copy(
                table_hbm.at[idx_v.at[pl.ds(0, _TP)]],
                rows_v.at[b],
                gsem.at[b],
            ).wait()

        def start_out(j, b):
            pltpu.async_copy(rows_v.at[b], out_hbm.at[base + j], osem.at[b])

        def wait_out(b):
            pltpu.make_async_copy(
                rows_v.at[b], out_hbm.at[base], osem.at[b]
            ).wait()

        start_gather(0, 0)
        for b in range(_NBUF):
            if b == _NBUF - 1:
                wait_out(0)
            start_gather(b + 1, (b + 1) % _NBUF)
            wait_gather(b)
            start_out(b, b)

        def group(g, _):
            for b in range(_NBUF):
                j = g * _NBUF + b
                wait_out((b + 1) % _NBUF)
                start_gather(j + 1, (b + 1) % _NBUF)
                wait_gather(b)
                start_out(j, b)
            return ()

        lax.fori_loop(1, n_groups - 1, group, (), unroll=False)

        for b in range(_NBUF):
            j = (n_groups - 1) * _NBUF + b
            if b != _NBUF - 1:
                wait_out((b + 1) % _NBUF)
                start_gather(j + 1, (b + 1) % _NBUF)
            wait_gather(b)
            start_out(j, b)

        for b in range(_NBUF):
            wait_out(b)

    return k


def kernel(token_ids, E):
    S, T = token_ids.shape
    idx = jnp.pad(token_ids.astype(jnp.int32), ((0, 0), (0, _TP - T)))
    out = _make_gather(S)(E, idx.reshape(-1))
    return out[:, :T, :]


# speedup vs baseline: 4.6554x; 4.6554x over previous
import functools

import jax
import jax.numpy as jnp
from jax import lax
from jax.experimental import pallas as pl
from jax.experimental.pallas import tpu as pltpu
from jax.experimental.pallas import tpu_sc as plsc

_NC = 2
_NS = 16
_NW = _NC * _NS

_D = 128
_NBUF = 4
_PIECES = 2


def _make_gather(S: int, T: int):
    assert S % _NW == 0
    s_per_w = S // _NW
    assert s_per_w % _NBUF == 0 and s_per_w // _NBUF >= 2
    n_groups = s_per_w // _NBUF
    mesh = plsc.VectorSubcoreMesh(core_axis_name="c", subcore_axis_name="s")

    @functools.partial(
        pl.kernel,
        mesh=mesh,
        out_type=jax.ShapeDtypeStruct((S, T, _D), jnp.float32),
        scratch_types=[
            pltpu.VMEM((s_per_w, T), jnp.int32),
            pltpu.VMEM((_NBUF, T, _D), jnp.float32),
            pltpu.SemaphoreType.DMA((_NBUF,)),
            pltpu.SemaphoreType.DMA((_NBUF,)),
        ],
    )
    def k(table_hbm, idx_hbm, out_hbm, idx_v, rows_v, gsem, osem):
        wid = lax.axis_index("s") * _NC + lax.axis_index("c")
        base = wid * s_per_w
        pltpu.sync_copy(idx_hbm.at[pl.ds(base, s_per_w)], idx_v)

        def start_gather(j, b):
            pltpu.async_copy(
                table_hbm.at[idx_v.at[j]], rows_v.at[b], gsem.at[b]
            )

        def wait_gather(b):
            pltpu.make_async_copy(
                table_hbm.at[idx_v.at[0]], rows_v.at[b], gsem.at[b]
            ).wait()

        def start_out(j, b):
            pltpu.async_copy(rows_v.at[b], out_hbm.at[base + j], osem.at[b])

        def wait_out(b):
            pltpu.make_async_copy(
                rows_v.at[b], out_hbm.at[base], osem.at[b]
            ).wait()

        start_gather(0, 0)
        for b in range(_NBUF):
            if b == _NBUF - 1:
                wait_out(0)
            start_gather(b + 1, (b + 1) % _NBUF)
            wait_gather(b)
            start_out(b, b)

        def group(g, _):
            for b in range(_NBUF):
                j = g * _NBUF + b
                wait_out((b + 1) % _NBUF)
                start_gather(j + 1, (b + 1) % _NBUF)
                wait_gather(b)
                start_out(j, b)
            return ()

        lax.fori_loop(1, n_groups - 1, group, (), unroll=False)

        for b in range(_NBUF):
            j = (n_groups - 1) * _NBUF + b
            if b != _NBUF - 1:
                wait_out((b + 1) % _NBUF)
                start_gather(j + 1, (b + 1) % _NBUF)
            wait_gather(b)
            start_out(j, b)

        for b in range(_NBUF):
            wait_out(b)

    return k


def kernel(token_ids, E):
    S, T = token_ids.shape
    idx = token_ids.astype(jnp.int32)
    ps = S // _PIECES
    f = _make_gather(ps, T)
    pieces = [
        f(E, lax.slice_in_dim(idx, i * ps, (i + 1) * ps, axis=0))
        for i in range(_PIECES)
    ]
    return jnp.concatenate(pieces, axis=0)


# speedup vs baseline: 7.8470x; 1.6856x over previous
import functools

import jax
import jax.numpy as jnp
from jax import lax
from jax.experimental import pallas as pl
from jax.experimental.pallas import tpu as pltpu
from jax.experimental.pallas import tpu_sc as plsc

_NC = 2
_NS = 16
_NW = _NC * _NS

_D = 128
_NBUF = 8
_PF = 2


def _make_gather(S: int, T: int):
    assert S % _NW == 0
    s_per_w = S // _NW
    assert s_per_w % _NBUF == 0 and s_per_w // _NBUF >= 2
    n_groups = s_per_w // _NBUF
    mesh = plsc.VectorSubcoreMesh(core_axis_name="c", subcore_axis_name="s")

    @functools.partial(
        pl.kernel,
        mesh=mesh,
        out_type=jax.ShapeDtypeStruct((S, T, _D), jnp.float32),
        scratch_types=[
            pltpu.VMEM((s_per_w, T), jnp.int32),
            pltpu.VMEM((_NBUF, T, _D), jnp.float32),
            pltpu.SemaphoreType.DMA((_NBUF,)),
            pltpu.SemaphoreType.DMA((_NBUF,)),
        ],
    )
    def k(table_hbm, idx_hbm, out_hbm, idx_v, rows_v, gsem, osem):
        wid = lax.axis_index("s") * _NC + lax.axis_index("c")
        base = wid * s_per_w
        pltpu.sync_copy(idx_hbm.at[pl.ds(base, s_per_w)], idx_v)

        def start_gather(j, b):
            pltpu.async_copy(
                table_hbm.at[idx_v.at[j]], rows_v.at[b], gsem.at[b]
            )

        def wait_gather(b):
            pltpu.make_async_copy(
                table_hbm.at[idx_v.at[0]], rows_v.at[b], gsem.at[b]
            ).wait()

        def start_out(j, b):
            pltpu.async_copy(rows_v.at[b], out_hbm.at[base + j], osem.at[b])

        def wait_out(b):
            pltpu.make_async_copy(
                rows_v.at[b], out_hbm.at[base], osem.at[b]
            ).wait()

        for p in range(_PF):
            start_gather(p, p)
        for b in range(_NBUF):
            nxt = b + _PF
            if nxt >= _NBUF:
                wait_out(nxt % _NBUF)
            start_gather(nxt, nxt % _NBUF)
            wait_gather(b)
            start_out(b, b)

        def group(g, _):
            for b in range(_NBUF):
                j = g * _NBUF + b
                wait_out((b + _PF) % _NBUF)
                start_gather(j + _PF, (b + _PF) % _NBUF)
                wait_gather(b)
                start_out(j, b)
            return ()

        lax.fori_loop(1, n_groups - 1, group, (), unroll=False)

        for b in range(_NBUF):
            j = (n_groups - 1) * _NBUF + b
            if b < _NBUF - _PF:
                wait_out((b + _PF) % _NBUF)
                start_gather(j + _PF, (b + _PF) % _NBUF)
            wait_gather(b)
            start_out(j, b)

        for b in range(_NBUF):
            wait_out(b)

    return k


def kernel(token_ids, E):
    S, T = token_ids.shape
    return _make_gather(S, T)(E, token_ids.astype(jnp.int32))


# speedup vs baseline: 7.9424x; 1.0121x over previous
import functools

import jax
import jax.numpy as jnp
from jax import lax
from jax.experimental import pallas as pl
from jax.experimental.pallas import tpu as pltpu
from jax.experimental.pallas import tpu_sc as plsc

_NC = 2
_NS = 16
_NW = _NC * _NS

_D = 128
_NBUF = 8
_PF = 4


def _make_gather(S: int, T: int):
    assert S % _NW == 0
    s_per_w = S // _NW
    assert s_per_w % _NBUF == 0 and s_per_w // _NBUF >= 2
    n_groups = s_per_w // _NBUF
    mesh = plsc.VectorSubcoreMesh(core_axis_name="c", subcore_axis_name="s")

    @functools.partial(
        pl.kernel,
        mesh=mesh,
        out_type=jax.ShapeDtypeStruct((S, T, _D), jnp.float32),
        scratch_types=[
            pltpu.VMEM((s_per_w, T), jnp.int32),
            pltpu.VMEM((_NBUF, T, _D), jnp.float32),
            pltpu.SemaphoreType.DMA((_NBUF,)),
            pltpu.SemaphoreType.DMA((_NBUF,)),
        ],
    )
    def k(table_hbm, idx_hbm, out_hbm, idx_v, rows_v, gsem, osem):
        wid = lax.axis_index("s") * _NC + lax.axis_index("c")
        base = wid * s_per_w
        pltpu.sync_copy(idx_hbm.at[pl.ds(base, s_per_w)], idx_v)

        def start_gather(j, b):
            pltpu.async_copy(
                table_hbm.at[idx_v.at[j]], rows_v.at[b], gsem.at[b]
            )

        def wait_gather(b):
            pltpu.make_async_copy(
                table_hbm.at[idx_v.at[0]], rows_v.at[b], gsem.at[b]
            ).wait()

        def start_out(j, b):
            pltpu.async_copy(rows_v.at[b], out_hbm.at[base + j], osem.at[b])

        def wait_out(b):
            pltpu.make_async_copy(
                rows_v.at[b], out_hbm.at[base], osem.at[b]
            ).wait()

        for p in range(_PF):
            start_gather(p, p)
        for b in range(_NBUF):
            nxt = b + _PF
            if nxt >= _NBUF:
                wait_out(nxt % _NBUF)
            start_gather(nxt, nxt % _NBUF)
            wait_gather(b)
            start_out(b, b)

        def group(g, _):
            for b in range(_NBUF):
                j = g * _NBUF + b
                wait_out((b + _PF) % _NBUF)
                start_gather(j + _PF, (b + _PF) % _NBUF)
                wait_gather(b)
                start_out(j, b)
            return ()

        lax.fori_loop(1, n_groups - 1, group, (), unroll=False)

        for b in range(_NBUF):
            j = (n_groups - 1) * _NBUF + b
            if b < _NBUF - _PF:
                wait_out((b + _PF) % _NBUF)
                start_gather(j + _PF, (b + _PF) % _NBUF)
            wait_gather(b)
            start_out(j, b)

        for b in range(_NBUF):
            wait_out(b)

    return k


def kernel(token_ids, E):
    S, T = token_ids.shape
    return _make_gather(S, T)(E, token_ids.astype(jnp.int32))
